# Initial kernel scaffold; baseline (speedup 1.0000x reference)
import jax
import jax.numpy as jnp
from jax.experimental import pallas as pl


def kernel(l, R):
    raise NotImplementedError("write your pallas kernel here")


# speedup vs baseline: 1.0812x; 1.0812x over previous
import functools

import jax
import jax.numpy as jnp
from jax import lax
from jax.experimental import pallas as pl
from jax.experimental.pallas import tpu as pltpu
from jax.experimental.pallas import tpu_sc as plsc

_N = 1000
_B = 16384
_NC = 2
_NS = 16
_NW = _NC * _NS
_BPW = _B // _NW
_CHUNK = 64
_NCH = _BPW // _CHUNK


def _gather_sc(l3, R):
    mesh = plsc.VectorSubcoreMesh(core_axis_name="c", subcore_axis_name="s")

    @functools.partial(
        pl.kernel,
        mesh=mesh,
        out_type=jax.ShapeDtypeStruct((_B, _N), jnp.float32),
        scratch_types=[
            pltpu.VMEM((_NCH, _CHUNK), jnp.int32),
            pltpu.VMEM((_CHUNK, _N), jnp.float32),
            pltpu.VMEM((_CHUNK, _N), jnp.float32),
            pltpu.SemaphoreType.DMA,
            pltpu.SemaphoreType.DMA,
        ],
        compiler_params=pltpu.CompilerParams(use_tc_tiling_on_sc=False),
    )
    def k(l_hbm, table_hbm, out_hbm, idx_v, buf0, buf1, sem0, sem1):
        wid = lax.axis_index("s") * _NC + lax.axis_index("c")
        base = wid * _BPW
        pltpu.sync_copy(l_hbm.at[wid], idx_v)
        bufs = (buf0, buf1)
        sems = (sem0, sem1)
        pending = pltpu.async_copy(table_hbm.at[idx_v.at[0]], buf0, sem0)
        for g in range(_NCH):
            nxt = None
            if g + 1 < _NCH:
                nxt = pltpu.async_copy(
                    table_hbm.at[idx_v.at[g + 1]],
                    bufs[(g + 1) % 2], sems[(g + 1) % 2])
            pending.wait()
            pltpu.sync_copy(bufs[g % 2],
                            out_hbm.at[pl.ds(base + g * _CHUNK, _CHUNK)])
            pending = nxt

    return k(l3, R)


def _penalty_body(r_ref, out_ref):
    x = r_ref[...]
    rows = lax.broadcasted_iota(jnp.int32, x.shape, 0)
    cols = lax.broadcasted_iota(jnp.int32, x.shape, 1)
    diag = jnp.where(rows == cols, x, jnp.float32(0.0))
    s = jnp.sum(x * x) - 2.0 * jnp.sum(diag) + jnp.float32(_N)
    out_ref[0, 0] = s / jnp.float32(_N * _N)


def _penalty_tc(R):
    out = pl.pallas_call(
        _penalty_body,
        out_shape=jax.ShapeDtypeStruct((1, 1), jnp.float32),
        out_specs=pl.BlockSpec(memory_space=pltpu.SMEM),
    )(R)
    return out.reshape(())


def kernel(l, R):
    l3 = l.astype(jnp.int32).reshape(_NW, _NCH, _CHUNK)
    laff = _gather_sc(l3, R)
    penalty = _penalty_tc(R)
    return (laff, penalty)


# speedup vs baseline: 1.5596x; 1.4424x over previous
import functools

import jax
import jax.numpy as jnp
from jax import lax
from jax.experimental import pallas as pl
from jax.experimental.pallas import tpu as pltpu
from jax.experimental.pallas import tpu_sc as plsc

_N = 1000
_NP = 1024
_BULK = 896
_TAIL = _N - _BULK
_B = 16384
_NC = 2
_NS = 16
_NW = _NC * _NS
_BPW = _B // _NW
_CHUNK = 32
_NCH = _BPW // _CHUNK


def _gather_sc(l2, Rp):
    mesh = plsc.VectorSubcoreMesh(core_axis_name="c", subcore_axis_name="s")

    @functools.partial(
        pl.kernel,
        mesh=mesh,
        out_type=jax.ShapeDtypeStruct((_B, _N), jnp.float32),
        scratch_types=[
            pltpu.VMEM((_BPW,), jnp.int32),
            pltpu.VMEM((_CHUNK, _NP), jnp.float32),
            pltpu.VMEM((_CHUNK, _NP), jnp.float32),
            pltpu.VMEM((_CHUNK, _TAIL), jnp.float32),
            pltpu.SemaphoreType.DMA,
            pltpu.SemaphoreType.DMA,
            pltpu.SemaphoreType.DMA,
        ],
    )
    def k(l_hbm, table_hbm, out_hbm, idx_v, buf0, buf1, tail_v,
          sem0, sem1, semw):
        wid = lax.axis_index("s") * _NC + lax.axis_index("c")
        base = wid * _BPW
        pltpu.sync_copy(l_hbm.at[wid], idx_v)
        bufs = (buf0, buf1)
        sems = (sem0, sem1)
        pending = pltpu.async_copy(
            table_hbm.at[idx_v.at[pl.ds(0, _CHUNK)]], buf0, sem0)
        for g in range(_NCH):
            nxt = None
            if g + 1 < _NCH:
                nxt = pltpu.async_copy(
                    table_hbm.at[idx_v.at[pl.ds((g + 1) * _CHUNK, _CHUNK)]],
                    bufs[(g + 1) % 2], sems[(g + 1) % 2])
            pending.wait()
            b = bufs[g % 2]
            rows = pl.ds(base + g * _CHUNK, _CHUNK)
            wcp = pltpu.async_copy(
                b.at[:, pl.ds(0, _BULK)], out_hbm.at[rows, pl.ds(0, _BULK)],
                semw)

            def repack_row(r, carry):
                for t in range(6):
                    tail_v[r, pl.ds(t * 16, 16)] = b[r, pl.ds(_BULK + t * 16, 16)]
                tail_v[r, pl.ds(_TAIL - 16, 16)] = b[r, pl.ds(_BULK + _TAIL - 16, 16)]
                return carry

            lax.fori_loop(0, _CHUNK, repack_row, 0)
            wcp.wait()
            pltpu.sync_copy(tail_v, out_hbm.at[rows, pl.ds(_BULK, _TAIL)])
            pending = nxt

    return k(l2, Rp)


def _penalty_body(r_ref, out_ref):
    x = r_ref[...]
    rows = lax.broadcasted_iota(jnp.int32, x.shape, 0)
    cols = lax.broadcasted_iota(jnp.int32, x.shape, 1)
    d = jnp.where(rows == cols, jnp.float32(1.0), jnp.float32(0.0)) - x
    col_sums = jnp.sum(d * d, axis=0)
    out_ref[0, 0] = jnp.sum(col_sums) / jnp.float32(_N * _N)


def _penalty_tc(R):
    out = pl.pallas_call(
        _penalty_body,
        out_shape=jax.ShapeDtypeStruct((1, 1), jnp.float32),
        out_specs=pl.BlockSpec(memory_space=pltpu.SMEM),
    )(R)
    return out.reshape(())


def kernel(l, R):
    l2 = l.astype(jnp.int32).reshape(_NW, _BPW)
    Rp = jnp.pad(R, ((0, 0), (0, _NP - _N)))
    laff = _gather_sc(l2, Rp)
    penalty = _penalty_tc(R)
    return (laff, penalty)


# speedup vs baseline: 2.8875x; 1.8514x over previous
import functools

import jax
import jax.numpy as jnp
from jax import lax
from jax.experimental import pallas as pl
from jax.experimental.pallas import tpu as pltpu
from jax.experimental.pallas import tpu_sc as plsc

_N = 1000
_NP = 1024
_B = 16384
_NC = 2
_NS = 16
_NW = _NC * _NS
_CG = 125
_RB = 128
_RBB = 32
_MAXJ = 4


def _gather_sc(l, Rt):
    mesh = plsc.VectorSubcoreMesh(core_axis_name="c", subcore_axis_name="s")

    @functools.partial(
        pl.kernel,
        mesh=mesh,
        out_type=jax.ShapeDtypeStruct((_CG, _RB, 8, 128), jnp.float32),
        scratch_types=[
            pltpu.VMEM((_B,), jnp.int32),
            pltpu.VMEM((_MAXJ * 8, _NP), jnp.float32),
            pltpu.VMEM((_RBB, 8, 128), jnp.float32),
            pltpu.VMEM((_RBB, 8, 128), jnp.float32),
            pltpu.SemaphoreType.DMA,
            pltpu.SemaphoreType.DMA,
            pltpu.SemaphoreType.DMA,
            pltpu.SemaphoreType.DMA,
            pltpu.SemaphoreType.DMA,
            pltpu.SemaphoreType.DMA,
        ],
        compiler_params=pltpu.CompilerParams(needs_layout_passes=False),
    )
    def k(l_hbm, rt_hbm, out_hbm, idx_v, rbuf, ob0, ob1,
          semr0, semr1, semr2, semr3, semw0, semw1):
        semrs = (semr0, semr1, semr2, semr3)
        wid = lax.axis_index("s") * _NC + lax.axis_index("c")
        cgs = [jnp.where(wid + j * _NW < _CG, wid + j * _NW,
                         wid + j * _NW - _CG) for j in range(_MAXJ)]
        rcps = [pltpu.async_copy(rt_hbm.at[pl.ds(cgs[j] * 8, 8)],
                                 rbuf.at[pl.ds(j * 8, 8)], semrs[j])
                for j in range(_MAXJ)]
        pltpu.sync_copy(l_hbm, idx_v)
        obs = (ob0, ob1)
        sems = (semw0, semw1)
        wcps = [None, None]
        nblk = _RB // _RBB
        for j in range(_MAXJ):
            rowvecs = [jnp.full((16,), j * 8 + c, jnp.int32) for c in range(8)]
            rcps[j].wait()
            for blk in range(nblk):
                slot = (j * nblk + blk) % 2
                ob = obs[slot]
                if wcps[slot] is not None:
                    wcps[slot].wait()

                @plsc.parallel_loop(0, _RBB)
                def fill_row(rb):
                    ivs = [idx_v[pl.ds((blk * _RBB + rb) * 128 + rv * 16, 16)]
                           for rv in range(8)]
                    pending = []
                    for rv in range(8):
                        for c in range(8):
                            pending.append((rv, c, plsc.load_gather(
                                rbuf, [rowvecs[c], ivs[rv]])))
                            if len(pending) > 3:
                                rv0, c0, g0 = pending.pop(0)
                                ob[rb, c0, pl.ds(rv0 * 16, 16)] = g0
                    for rv0, c0, g0 in pending:
                        ob[rb, c0, pl.ds(rv0 * 16, 16)] = g0
                wcps[slot] = pltpu.async_copy(
                    ob, out_hbm.at[cgs[j], pl.ds(blk * _RBB, _RBB)],
                    sems[slot])

        for slot in range(2):
            if wcps[slot] is not None:
                wcps[slot].wait()

    return k(l, Rt)


def _penalty_body(r_ref, out_ref):
    x = r_ref[...]
    rows = lax.broadcasted_iota(jnp.int32, x.shape, 0)
    cols = lax.broadcasted_iota(jnp.int32, x.shape, 1)
    d = jnp.where(rows == cols, jnp.float32(1.0), jnp.float32(0.0)) - x
    col_sums = jnp.sum(d * d, axis=0)
    out_ref[0, 0] = jnp.sum(col_sums) / jnp.float32(_N * _N)


def _penalty_tc(R):
    out = pl.pallas_call(
        _penalty_body,
        out_shape=jax.ShapeDtypeStruct((1, 1), jnp.float32),
        out_specs=pl.BlockSpec(memory_space=pltpu.SMEM),
    )(R)
    return out.reshape(())


def kernel(l, R):
    li = l.astype(jnp.int32)
    Rt = jnp.pad(R.T, ((0, 0), (0, _NP - _N)))
    out4 = _gather_sc(li, Rt)
    laff = jnp.transpose(out4, (1, 3, 0, 2)).reshape(_B, _N)
    penalty = _penalty_tc(R)
    return (laff, penalty)


# speedup vs baseline: 2.8925x; 1.0017x over previous
import functools

import jax
import jax.numpy as jnp
from jax import lax
from jax.experimental import pallas as pl
from jax.experimental.pallas import tpu as pltpu
from jax.experimental.pallas import tpu_sc as plsc

_N = 1000
_NP = 1024
_B = 16384
_NC = 2
_NS = 16
_NW = _NC * _NS
_CG = 125
_RB = 128
_RBB = 32
_MAXJ = 4


def _gather_sc(l, Rt):
    mesh = plsc.VectorSubcoreMesh(core_axis_name="c", subcore_axis_name="s")

    @functools.partial(
        pl.kernel,
        mesh=mesh,
        out_type=jax.ShapeDtypeStruct((_CG, _RB, 8, 128), jnp.float32),
        scratch_types=[
            pltpu.VMEM((_B,), jnp.int32),
            pltpu.VMEM((_MAXJ * 8, _NP), jnp.float32),
            pltpu.VMEM((_RBB, 8, 128), jnp.float32),
            pltpu.VMEM((_RBB, 8, 128), jnp.float32),
            pltpu.SemaphoreType.DMA,
            pltpu.SemaphoreType.DMA,
            pltpu.SemaphoreType.DMA,
            pltpu.SemaphoreType.DMA,
            pltpu.SemaphoreType.DMA,
            pltpu.SemaphoreType.DMA,
        ],
        compiler_params=pltpu.CompilerParams(needs_layout_passes=False),
    )
    def k(l_hbm, rt_hbm, out_hbm, idx_v, rbuf, ob0, ob1,
          semr0, semr1, semr2, semr3, semw0, semw1):
        semrs = (semr0, semr1, semr2, semr3)
        wid = lax.axis_index("s") * _NC + lax.axis_index("c")
        cgs = [jnp.where(wid + j * _NW < _CG, wid + j * _NW,
                         wid + j * _NW - _CG) for j in range(_MAXJ)]
        rcps = [pltpu.async_copy(rt_hbm.at[pl.ds(cgs[j] * 8, 8)],
                                 rbuf.at[pl.ds(j * 8, 8)], semrs[j])
                for j in range(_MAXJ)]
        pltpu.sync_copy(l_hbm, idx_v)
        obs = (ob0, ob1)
        sems = (semw0, semw1)
        wcps = [None, None]
        nblk = _RB // _RBB
        for j in range(_MAXJ):
            rowvecs = [jnp.full((16,), j * 8 + c, jnp.int32) for c in range(8)]
            rcps[j].wait()
            for blk in range(nblk):
                slot = (j * nblk + blk) % 2
                ob = obs[slot]
                if wcps[slot] is not None:
                    wcps[slot].wait()

                @plsc.parallel_loop(0, _RBB)
                def fill_row(rb):
                    base = (blk * _RBB + rb) * 128
                    iv = idx_v[pl.ds(base, 16)]
                    pending = []
                    for rv in range(8):
                        iv_cur = iv
                        if rv + 1 < 8:
                            iv = idx_v[pl.ds(base + (rv + 1) * 16, 16)]
                        for c in range(8):
                            pending.append((rv, c, plsc.load_gather(
                                rbuf, [rowvecs[c], iv_cur])))
                            if len(pending) > 3:
                                rv0, c0, g0 = pending.pop(0)
                                ob[rb, c0, pl.ds(rv0 * 16, 16)] = g0
                    for rv0, c0, g0 in pending:
                        ob[rb, c0, pl.ds(rv0 * 16, 16)] = g0
                wcps[slot] = pltpu.async_copy(
                    ob, out_hbm.at[cgs[j], pl.ds(blk * _RBB, _RBB)],
                    sems[slot])

        for slot in range(2):
            if wcps[slot] is not None:
                wcps[slot].wait()

    return k(l, Rt)


def _penalty_body(r_ref, out_ref):
    x = r_ref[...]
    rows = lax.broadcasted_iota(jnp.int32, x.shape, 0)
    cols = lax.broadcasted_iota(jnp.int32, x.shape, 1)
    d = jnp.where(rows == cols, jnp.float32(1.0), jnp.float32(0.0)) - x
    col_sums = jnp.sum(d * d, axis=0)
    out_ref[0, 0] = jnp.sum(col_sums) / jnp.float32(_N * _N)


def _penalty_tc(R):
    out = pl.pallas_call(
        _penalty_body,
        out_shape=jax.ShapeDtypeStruct((1, 1), jnp.float32),
        out_specs=pl.BlockSpec(memory_space=pltpu.SMEM),
    )(R)
    return out.reshape(())


def kernel(l, R):
    li = l.astype(jnp.int32)
    Rt = jnp.pad(R.T, ((0, 0), (0, _NP - _N)))
    out4 = _gather_sc(li, Rt)
    laff = jnp.transpose(out4, (1, 3, 0, 2)).reshape(_B, _N)
    penalty = _penalty_tc(R)
    return (laff, penalty)
